# baseline (speedup 1.0000x reference)
import functools

import jax
import jax.numpy as jnp
from jax import lax
from jax.experimental import pallas as pl
from jax.experimental.pallas import tpu as pltpu
from jax.experimental.pallas import tpu_sc as plsc

_ROWS = 6
_D = 64
_L = 16

_mesh = plsc.VectorSubcoreMesh(
    core_axis_name="c", subcore_axis_name="s", num_cores=1, num_subcores=1
)


@functools.partial(
    pl.kernel,
    out_type=jax.ShapeDtypeStruct((_ROWS, _D), jnp.float32),
    mesh=_mesh,
    scratch_types=[
        pltpu.VMEM((_ROWS + 2, _D), jnp.float32),
        pltpu.VMEM((_ROWS, _D), jnp.float32),
    ],
)
def _shifted_add(x_hbm, out_hbm, xbuf, obuf):
    cid = lax.axis_index("c")
    sid = lax.axis_index("s")

    @pl.when((cid == 0) & (sid == 0))
    def _():
        pltpu.sync_copy(x_hbm, xbuf)
        for r in range(_ROWS):
            for c in range(0, _D, _L):
                obuf[r, pl.ds(c, _L)] = (
                    xbuf[r, pl.ds(c, _L)] + xbuf[r + 1, pl.ds(c, _L)]
                )
        pltpu.sync_copy(obuf, out_hbm)


def _extract_head(x_ref, head_ref):
    head_ref[...] = x_ref[...]


def kernel(x):
    head = pl.pallas_call(
        _extract_head,
        out_shape=jax.ShapeDtypeStruct((_ROWS + 2, _D), jnp.float32),
        grid=(1,),
        in_specs=[pl.BlockSpec((_ROWS + 2, _D), lambda i: (0, 0))],
        out_specs=pl.BlockSpec((_ROWS + 2, _D), lambda i: (0, 0)),
    )(x)
    return _shifted_add(head).reshape(3, 2, _D)


# speedup vs baseline: 18.1437x; 18.1437x over previous
import functools

import jax
import jax.numpy as jnp
from jax import lax
from jax.experimental import pallas as pl
from jax.experimental.pallas import tpu as pltpu
from jax.experimental.pallas import tpu_sc as plsc

_ROWS = 6
_D = 64
_L = 16

_mesh = plsc.VectorSubcoreMesh(
    core_axis_name="c", subcore_axis_name="s", num_cores=1, num_subcores=1
)


@functools.partial(
    pl.kernel,
    out_type=jax.ShapeDtypeStruct((_ROWS, _D), jnp.float32),
    mesh=_mesh,
    scratch_types=[
        pltpu.VMEM((_ROWS + 2, _D), jnp.float32),
        pltpu.VMEM((_ROWS, _D), jnp.float32),
    ],
)
def _shifted_add(x_hbm, out_hbm, xbuf, obuf):
    pltpu.sync_copy(x_hbm, xbuf)
    for r in range(_ROWS):
        for c in range(0, _D, _L):
            obuf[r, pl.ds(c, _L)] = (
                xbuf[r, pl.ds(c, _L)] + xbuf[r + 1, pl.ds(c, _L)]
            )
    pltpu.sync_copy(obuf, out_hbm)


def kernel(x):
    head = jax.lax.slice(x, (0, 0), (_ROWS + 2, _D))
    return _shifted_add(head).reshape(3, 2, _D)


# speedup vs baseline: 19.2874x; 1.0630x over previous
import functools

import jax
import jax.numpy as jnp
from jax import lax
from jax.experimental import pallas as pl
from jax.experimental.pallas import tpu as pltpu
from jax.experimental.pallas import tpu_sc as plsc

_ROWS = 6
_D = 64
_L = 16

_mesh = plsc.VectorSubcoreMesh(
    core_axis_name="c", subcore_axis_name="s", num_cores=1, num_subcores=1
)


@functools.partial(
    pl.kernel,
    out_type=jax.ShapeDtypeStruct((3, 2, _D), jnp.float32),
    mesh=_mesh,
    scratch_types=[
        pltpu.VMEM((_ROWS + 2, _D), jnp.float32),
        pltpu.VMEM((3, 2, _D), jnp.float32),
    ],
)
def _shifted_add(x_hbm, out_hbm, xbuf, obuf):
    pltpu.sync_copy(x_hbm, xbuf)
    for r in range(_ROWS):
        for c in range(0, _D, _L):
            obuf[r // 2, r % 2, pl.ds(c, _L)] = (
                xbuf[r, pl.ds(c, _L)] + xbuf[r + 1, pl.ds(c, _L)]
            )
    pltpu.sync_copy(obuf, out_hbm)


def kernel(x):
    head = jax.lax.slice(x, (0, 0), (_ROWS + 2, _D))
    return _shifted_add(head)


# speedup vs baseline: 19.3650x; 1.0040x over previous
import functools

import jax
import jax.numpy as jnp
from jax.experimental import pallas as pl
from jax.experimental.pallas import tpu as pltpu
from jax.experimental.pallas import tpu_sc as plsc

_ROWS = 6
_D = 64
_L = 16

_mesh = plsc.VectorSubcoreMesh(
    core_axis_name="c", subcore_axis_name="s", num_cores=1, num_subcores=1
)


@functools.partial(
    pl.kernel,
    out_type=jax.ShapeDtypeStruct((3, 2, _D), jnp.float32),
    mesh=_mesh,
    scratch_types=[
        pltpu.VMEM((_ROWS + 2, _D), jnp.float32),
        pltpu.VMEM((3, 2, _D), jnp.float32),
    ],
)
def _shifted_add(x_hbm, out_hbm, xbuf, obuf):
    pltpu.sync_copy(x_hbm, xbuf)
    for r in range(_ROWS):
        for c in range(0, _D, _L):
            obuf[r // 2, r % 2, pl.ds(c, _L)] = (
                xbuf[r, pl.ds(c, _L)] + xbuf[r + 1, pl.ds(c, _L)]
            )
    pltpu.sync_copy(obuf, out_hbm)


def kernel(x):
    head = jax.lax.slice(x, (0, 0), (_ROWS + 2, _D))
    return _shifted_add(head)


# speedup vs baseline: 19.3949x; 1.0015x over previous
import functools

import jax
import jax.numpy as jnp
from jax.experimental import pallas as pl
from jax.experimental.pallas import tpu as pltpu
from jax.experimental.pallas import tpu_sc as plsc

_ROWS = 6
_D = 64
_L = 16

_mesh = plsc.VectorSubcoreMesh(
    core_axis_name="c", subcore_axis_name="s", num_cores=1, num_subcores=1
)


@functools.partial(
    pl.kernel,
    out_type=jax.ShapeDtypeStruct((3, 2, _D), jnp.float32),
    mesh=_mesh,
    scratch_types=[
        pltpu.VMEM(((_ROWS + 2) * _D,), jnp.float32),
        pltpu.VMEM((3, 2, _D), jnp.float32),
    ],
)
def _shifted_add(x_hbm, out_hbm, xbuf, obuf):
    pltpu.sync_copy(x_hbm, xbuf)
    for r in range(_ROWS):
        for c in range(0, _D, _L):
            obuf[r // 2, r % 2, pl.ds(c, _L)] = (
                xbuf[pl.ds(r * _D + c, _L)] + xbuf[pl.ds((r + 1) * _D + c, _L)]
            )
    pltpu.sync_copy(obuf, out_hbm)


def kernel(x):
    head = jax.lax.slice(x, (0, 0), (_ROWS + 2, _D))
    return _shifted_add(head.reshape((_ROWS + 2) * _D))
